# Initial kernel scaffold; baseline (speedup 1.0000x reference)
import functools

import jax
import jax.numpy as jnp
from jax import lax
from jax.experimental import pallas as pl
from jax.experimental.pallas import tpu as pltpu
from jax.experimental.pallas import tpu_sc as plsc

_NC = 2
_NS = 16
_NW = _NC * _NS
_L = 16


def _emb_kernel(n_total, vocab, d, chunk, ids_hbm, table_hbm, out_hbm,
                idx_v, rows_v, sem):
    wid = lax.axis_index("s") * _NC + lax.axis_index("c")
    per_w = n_total // _NW
    base = wid * per_w
    n_ch = per_w // chunk

    def chunk_body(j, _):
        off = base + j * chunk
        pltpu.sync_copy(ids_hbm.at[pl.ds(off, chunk)], idx_v)

        def clamp_body(i, _):
            v = idx_v[pl.ds(i * _L, _L)]
            idx_v[pl.ds(i * _L, _L)] = jnp.where(v >= vocab, 0, v)
            return 0

        lax.fori_loop(0, chunk // _L, clamp_body, 0, unroll=4)

        pltpu.async_copy(table_hbm.at[idx_v], rows_v, sem).wait()
        pltpu.sync_copy(rows_v, out_hbm.at[pl.ds(off, chunk)])
        return 0

    lax.fori_loop(0, n_ch, chunk_body, 0)


def kernel(input_ids, table):
    b, s = input_ids.shape
    v, d = table.shape
    n = b * s
    ids = input_ids.reshape(n)

    chunk = 1024
    assert n % (_NW * chunk) == 0

    mesh = plsc.VectorSubcoreMesh(core_axis_name="c", subcore_axis_name="s",
                                  num_cores=_NC, num_subcores=_NS)
    run = pl.kernel(
        functools.partial(_emb_kernel, n, v, d, chunk),
        out_type=jax.ShapeDtypeStruct((n, d), jnp.float32),
        mesh=mesh,
        scratch_types=[
            pltpu.VMEM((chunk,), jnp.int32),
            pltpu.VMEM((chunk, d), jnp.float32),
            pltpu.SemaphoreType.DMA,
        ],
    )
    out = run(ids, table)
    return out.reshape(b, s, d)


# speedup vs baseline: 1.8711x; 1.8711x over previous
import functools

import jax
import jax.numpy as jnp
from jax import lax
from jax.experimental import pallas as pl
from jax.experimental.pallas import tpu as pltpu
from jax.experimental.pallas import tpu_sc as plsc

_NC = 2
_NS = 16
_NW = _NC * _NS
_L = 16
_NBUF = 2


def _emb_kernel(n_total, vocab, d, chunk, ids_hbm, table_hbm, out_hbm,
                idx_v, rows_v, gsems):
    wid = lax.axis_index("s") * _NC + lax.axis_index("c")
    per_w = n_total // _NW
    base = wid * per_w
    n_ch = per_w // chunk

    def stage_and_fire(j, b):
        off = base + j * chunk
        pltpu.sync_copy(ids_hbm.at[pl.ds(off, chunk)], idx_v.at[b])

        def clamp_body(i, _):
            v = idx_v[b, pl.ds(i * _L, _L)]
            idx_v[b, pl.ds(i * _L, _L)] = jnp.where(v >= vocab, 0, v)
            return 0

        lax.fori_loop(0, chunk // _L, clamp_body, 0, unroll=4)
        pltpu.async_copy(table_hbm.at[idx_v.at[b]], rows_v.at[b], gsems[b])

    for b in range(_NBUF):
        stage_and_fire(b, b)

    def outer(k, _):
        j0 = k * _NBUF
        for b in range(_NBUF):
            j = j0 + b
            pltpu.make_async_copy(table_hbm.at[idx_v.at[b]], rows_v.at[b],
                                  gsems[b]).wait()
            pltpu.sync_copy(rows_v.at[b], out_hbm.at[pl.ds(base + j * chunk,
                                                           chunk)])
            nj = j + _NBUF

            @pl.when(nj < n_ch)
            def _():
                stage_and_fire(nj, b)

        return 0

    lax.fori_loop(0, n_ch // _NBUF, outer, 0)


def kernel(input_ids, table):
    b, s = input_ids.shape
    v, d = table.shape
    n = b * s
    ids = input_ids.reshape(n)

    chunk = 800
    assert n % (_NW * chunk * _NBUF) == 0

    mesh = plsc.VectorSubcoreMesh(core_axis_name="c", subcore_axis_name="s",
                                  num_cores=_NC, num_subcores=_NS)
    run = pl.kernel(
        functools.partial(_emb_kernel, n, v, d, chunk),
        out_type=jax.ShapeDtypeStruct((n, d), jnp.float32),
        mesh=mesh,
        scratch_types=[
            pltpu.VMEM((_NBUF, chunk), jnp.int32),
            pltpu.VMEM((_NBUF, chunk, d), jnp.float32),
            [pltpu.SemaphoreType.DMA] * _NBUF,
        ],
        compiler_params=pltpu.CompilerParams(use_tc_tiling_on_sc=False),
    )
    out = run(ids, table)
    return out.reshape(b, s, d)
